# Initial kernel scaffold; baseline (speedup 1.0000x reference)
import jax
import jax.numpy as jnp
from jax.experimental import pallas as pl


def kernel(boxes, scores):
    raise NotImplementedError("write your pallas kernel here")


# speedup vs baseline: 16.1968x; 16.1968x over previous
import functools

import jax
import jax.numpy as jnp
from jax import lax
from jax.experimental import pallas as pl
from jax.experimental.pallas import tpu as pltpu

N = 5000
BLK = 128
NPAD = 5120
NBLK = NPAD // BLK
IOU_THRESH = 0.5


def _nms_block_kernel(bT_ref, bC_ref, sC_ref, out_ref, supacc_ref, loc_ref):
    k = pl.program_id(0)

    @pl.when(k == 0)
    def _init():
        supacc_ref[...] = jnp.zeros_like(supacc_ref)

    blk = bT_ref[...]
    y1r, x1r, y2r, x2r = (blk[:, 0:1], blk[:, 1:2], blk[:, 2:3], blk[:, 3:4])
    allc = bC_ref[...]
    y1c, x1c, y2c, x2c = (allc[0:1, :], allc[1:2, :], allc[2:3, :], allc[3:4, :])

    area_r = (y2r - y1r) * (x2r - x1r)
    area_c = (y2c - y1c) * (x2c - x1c)

    yy1 = jnp.maximum(y1r, y1c)
    xx1 = jnp.maximum(x1r, x1c)
    yy2 = jnp.minimum(y2r, y2c)
    xx2 = jnp.minimum(x2r, x2c)
    ih = jnp.maximum(yy2 - yy1, 0.0)
    iw = jnp.maximum(xx2 - xx1, 0.0)
    inter = ih * iw
    union = area_r + area_c - inter
    iou = inter / (union + 1e-8)
    supf = (iou > IOU_THRESH).astype(jnp.float32)

    cols = pl.ds(k * BLK, BLK)
    y1b, x1b = bC_ref[0:1, cols], bC_ref[1:2, cols]
    y2b, x2b = bC_ref[2:3, cols], bC_ref[3:4, cols]
    area_b = (y2b - y1b) * (x2b - x1b)
    lyy1 = jnp.maximum(y1r, y1b)
    lxx1 = jnp.maximum(x1r, x1b)
    lyy2 = jnp.minimum(y2r, y2b)
    lxx2 = jnp.minimum(x2r, x2b)
    lih = jnp.maximum(lyy2 - lyy1, 0.0)
    liw = jnp.maximum(lxx2 - lxx1, 0.0)
    linter = lih * liw
    lunion = area_r + area_b - linter
    liou = linter / (lunion + 1e-8)
    loc_ref[...] = (liou > IOU_THRESH).astype(jnp.float32)

    sup_prev = supacc_ref[0:1, cols]
    kb0 = 1.0 - sup_prev

    lane = lax.broadcasted_iota(jnp.int32, (1, BLK), 1)

    def body(i, kb):
        row = loc_ref[pl.ds(i, 1), :]
        kb_i = jnp.max(jnp.where(lane == i, kb, 0.0))
        sup_i = row * kb_i * (lane > i).astype(jnp.float32)
        return kb * (1.0 - sup_i)

    kb = lax.fori_loop(0, BLK, body, kb0)

    hits = lax.dot_general(
        kb, supf, (((1,), (0,)), ((), ())), preferred_element_type=jnp.float32
    )
    supacc_ref[...] = jnp.maximum(supacc_ref[...], (hits > 0.0).astype(jnp.float32))

    bcols = bC_ref[:, cols]
    srow = sC_ref[...]
    out_ref[...] = jnp.concatenate(
        [bcols * kb, srow * kb, jnp.zeros((3, BLK), jnp.float32)], axis=0
    )


@jax.jit
def kernel(boxes, scores):
    order = jnp.argsort(-scores)
    b = boxes[order]
    s = scores[order]
    bT = jnp.zeros((NPAD, 4), jnp.float32).at[:N].set(b)
    bC = bT.T
    sC = jnp.zeros((1, NPAD), jnp.float32).at[0, :N].set(s)

    outT = pl.pallas_call(
        _nms_block_kernel,
        grid=(NBLK,),
        in_specs=[
            pl.BlockSpec((BLK, 4), lambda k: (k, 0)),
            pl.BlockSpec((4, NPAD), lambda k: (0, 0)),
            pl.BlockSpec((1, BLK), lambda k: (0, k)),
        ],
        out_specs=pl.BlockSpec((8, BLK), lambda k: (0, k)),
        out_shape=jax.ShapeDtypeStruct((8, NPAD), jnp.float32),
        scratch_shapes=[
            pltpu.VMEM((1, NPAD), jnp.float32),
            pltpu.VMEM((BLK, BLK), jnp.float32),
        ],
    )(bT, bC, sC)

    return outT[:5, :N].T


# speedup vs baseline: 43.2914x; 2.6728x over previous
import jax
import jax.numpy as jnp
from jax import lax
from jax.experimental import pallas as pl
from jax.experimental.pallas import tpu as pltpu

N = 5000
BLK = 128
NPAD = 5120
NBLK = NPAD // BLK
IOU_THRESH = 0.5


def _nms_block_kernel(bT_ref, bC_ref, sC_ref, out_ref, keep_ref):
    k = pl.program_id(0)

    @pl.when(k == 0)
    def _init():
        keep_ref[...] = jnp.zeros_like(keep_ref)

    cols = pl.ds(k * BLK, BLK)
    y1b, x1b = bC_ref[0:1, cols], bC_ref[1:2, cols]
    y2b, x2b = bC_ref[2:3, cols], bC_ref[3:4, cols]
    area_b = (y2b - y1b) * (x2b - x1b)

    allb = bT_ref[...]
    y1a, x1a = allb[:, 0:1], allb[:, 1:2]
    y2a, x2a = allb[:, 2:3], allb[:, 3:4]
    area_a = (y2a - y1a) * (x2a - x1a)

    yy1 = jnp.maximum(y1a, y1b)
    xx1 = jnp.maximum(x1a, x1b)
    yy2 = jnp.minimum(y2a, y2b)
    xx2 = jnp.minimum(x2a, x2b)
    ih = jnp.maximum(yy2 - yy1, 0.0)
    iw = jnp.maximum(xx2 - xx1, 0.0)
    inter = ih * iw
    union = area_a + area_b - inter
    iou = inter / (union + 1e-8)
    supT = (iou > IOU_THRESH).astype(jnp.float32)

    hits = lax.dot_general(
        keep_ref[...], supT, (((1,), (0,)), ((), ())),
        preferred_element_type=jnp.float32,
    )
    kb0 = (hits == 0.0).astype(jnp.float32)

    rb = bT_ref[pl.ds(k * BLK, BLK), :]
    y1r, x1r = rb[:, 0:1], rb[:, 1:2]
    y2r, x2r = rb[:, 2:3], rb[:, 3:4]
    area_r = (y2r - y1r) * (x2r - x1r)
    byy1 = jnp.maximum(y1r, y1b)
    bxx1 = jnp.maximum(x1r, x1b)
    byy2 = jnp.minimum(y2r, y2b)
    bxx2 = jnp.minimum(x2r, x2b)
    bih = jnp.maximum(byy2 - byy1, 0.0)
    biw = jnp.maximum(bxx2 - bxx1, 0.0)
    binter = bih * biw
    bunion = area_r + area_b - binter
    biou = binter / (bunion + 1e-8)
    rowi = lax.broadcasted_iota(jnp.int32, (BLK, BLK), 0)
    coli = lax.broadcasted_iota(jnp.int32, (BLK, BLK), 1)
    locU = ((biou > IOU_THRESH) & (coli > rowi)).astype(jnp.float32)

    def cond(c):
        return jnp.logical_not(c[1])

    def body(c):
        kb, _ = c
        h = lax.dot_general(
            kb, locU, (((1,), (0,)), ((), ())),
            preferred_element_type=jnp.float32,
        )
        kb2 = kb0 * (h == 0.0).astype(jnp.float32)
        return kb2, jnp.all(kb2 == kb)

    kb, _ = lax.while_loop(cond, body, (kb0, False))

    keep_ref[0:1, cols] = kb

    bcols = bC_ref[:, cols]
    srow = sC_ref[...]
    out_ref[...] = jnp.concatenate(
        [bcols * kb, srow * kb, jnp.zeros((3, BLK), jnp.float32)], axis=0
    )


@jax.jit
def kernel(boxes, scores):
    order = jnp.argsort(-scores)
    b = boxes[order]
    s = scores[order]
    bT = jnp.zeros((NPAD, 4), jnp.float32).at[:N].set(b)
    bC = bT.T
    sC = jnp.zeros((1, NPAD), jnp.float32).at[0, :N].set(s)

    outT = pl.pallas_call(
        _nms_block_kernel,
        grid=(NBLK,),
        in_specs=[
            pl.BlockSpec((NPAD, 4), lambda k: (0, 0)),
            pl.BlockSpec((4, NPAD), lambda k: (0, 0)),
            pl.BlockSpec((1, BLK), lambda k: (0, k)),
        ],
        out_specs=pl.BlockSpec((8, BLK), lambda k: (0, k)),
        out_shape=jax.ShapeDtypeStruct((8, NPAD), jnp.float32),
        scratch_shapes=[
            pltpu.VMEM((1, NPAD), jnp.float32),
        ],
    )(bT, bC, sC)

    return outT[:5, :N].T


# speedup vs baseline: 51.2148x; 1.1830x over previous
import jax
import jax.numpy as jnp
from jax import lax
from jax.experimental import pallas as pl
from jax.experimental.pallas import tpu as pltpu

N = 5000
BLK = 128
NPAD = 5120
NBLK = NPAD // BLK
IOU_THRESH = 0.5


CH = 512
CPB = CH // BLK


def _nms_block_kernel(bT_ref, bC_ref, sC_ref, out_ref, keep_ref):
    k = pl.program_id(0)

    @pl.when(k == 0)
    def _init():
        keep_ref[...] = jnp.zeros_like(keep_ref)

    cols = pl.ds(k * BLK, BLK)
    y1b, x1b = bC_ref[0:1, cols], bC_ref[1:2, cols]
    y2b, x2b = bC_ref[2:3, cols], bC_ref[3:4, cols]
    area_b = (y2b - y1b) * (x2b - x1b)

    def chunk_body(m, hits):
        rows = pl.ds(m * CH, CH)
        cb = bT_ref[rows, :]
        y1a, x1a = cb[:, 0:1], cb[:, 1:2]
        y2a, x2a = cb[:, 2:3], cb[:, 3:4]
        area_a = (y2a - y1a) * (x2a - x1a)
        yy1 = jnp.maximum(y1a, y1b)
        xx1 = jnp.maximum(x1a, x1b)
        yy2 = jnp.minimum(y2a, y2b)
        xx2 = jnp.minimum(x2a, x2b)
        ih = jnp.maximum(yy2 - yy1, 0.0)
        iw = jnp.maximum(xx2 - xx1, 0.0)
        inter = ih * iw
        union = area_a + area_b - inter
        iou = inter / (union + 1e-8)
        supT = (iou > IOU_THRESH).astype(jnp.float32)
        return hits + lax.dot_general(
            keep_ref[0:1, rows], supT, (((1,), (0,)), ((), ())),
            preferred_element_type=jnp.float32,
        )

    hits = lax.fori_loop(
        0, k // CPB + 1, chunk_body, jnp.zeros((1, BLK), jnp.float32)
    )
    kb0 = (hits == 0.0).astype(jnp.float32)

    rb = bT_ref[pl.ds(k * BLK, BLK), :]
    y1r, x1r = rb[:, 0:1], rb[:, 1:2]
    y2r, x2r = rb[:, 2:3], rb[:, 3:4]
    area_r = (y2r - y1r) * (x2r - x1r)
    byy1 = jnp.maximum(y1r, y1b)
    bxx1 = jnp.maximum(x1r, x1b)
    byy2 = jnp.minimum(y2r, y2b)
    bxx2 = jnp.minimum(x2r, x2b)
    bih = jnp.maximum(byy2 - byy1, 0.0)
    biw = jnp.maximum(bxx2 - bxx1, 0.0)
    binter = bih * biw
    bunion = area_r + area_b - binter
    biou = binter / (bunion + 1e-8)
    rowi = lax.broadcasted_iota(jnp.int32, (BLK, BLK), 0)
    coli = lax.broadcasted_iota(jnp.int32, (BLK, BLK), 1)
    locU = ((biou > IOU_THRESH) & (coli > rowi)).astype(jnp.float32)

    def cond(c):
        return jnp.logical_not(c[1])

    def body(c):
        kb, _ = c
        h = lax.dot_general(
            kb, locU, (((1,), (0,)), ((), ())),
            preferred_element_type=jnp.float32,
        )
        kb2 = kb0 * (h == 0.0).astype(jnp.float32)
        return kb2, jnp.all(kb2 == kb)

    kb, _ = lax.while_loop(cond, body, (kb0, False))

    keep_ref[0:1, cols] = kb

    bcols = bC_ref[:, cols]
    srow = sC_ref[...]
    out_ref[...] = jnp.concatenate(
        [bcols * kb, srow * kb, jnp.zeros((3, BLK), jnp.float32)], axis=0
    )


@jax.jit
def kernel(boxes, scores):
    order = jnp.argsort(-scores)
    b = boxes[order]
    s = scores[order]
    bT = jnp.zeros((NPAD, 4), jnp.float32).at[:N].set(b)
    bC = bT.T
    sC = jnp.zeros((1, NPAD), jnp.float32).at[0, :N].set(s)

    outT = pl.pallas_call(
        _nms_block_kernel,
        grid=(NBLK,),
        in_specs=[
            pl.BlockSpec((NPAD, 4), lambda k: (0, 0)),
            pl.BlockSpec((4, NPAD), lambda k: (0, 0)),
            pl.BlockSpec((1, BLK), lambda k: (0, k)),
        ],
        out_specs=pl.BlockSpec((8, BLK), lambda k: (0, k)),
        out_shape=jax.ShapeDtypeStruct((8, NPAD), jnp.float32),
        scratch_shapes=[
            pltpu.VMEM((1, NPAD), jnp.float32),
        ],
    )(bT, bC, sC)

    return outT[:5, :N].T


# speedup vs baseline: 52.5622x; 1.0263x over previous
import functools

import jax
import jax.numpy as jnp
from jax import lax
from jax.experimental import pallas as pl
from jax.experimental.pallas import tpu as pltpu
from jax.experimental.pallas import tpu_sc as plsc

N = 5000
BLK = 128
NPAD = 5120
NBLK = NPAD // BLK
IOU_THRESH = 0.5

NW = 32
BPW = NPAD // NW
SUB = 80
NSUB = BPW // SUB


def _sc_stage_kernel(
    flat_hbm, scores_hbm, idx_hbm, ordp_hbm,
    bC_hbm, sC_hbm,
    idx_v, val_v, ord_v, sem,
):
    w = lax.axis_index("s") * 2 + lax.axis_index("c")
    base = w * BPW

    pltpu.sync_copy(ordp_hbm.at[pl.ds(base, BPW)], ord_v)
    for t in range(NSUB):
        pltpu.async_copy(
            scores_hbm.at[ord_v.at[pl.ds(t * SUB, SUB)]],
            val_v.at[pl.ds(t * SUB, SUB)],
            sem,
        ).wait()
    pltpu.sync_copy(val_v, sC_hbm.at[pl.ds(base, BPW)])

    for c in range(4):
        pltpu.sync_copy(idx_hbm.at[pl.ds(c * NPAD + base, BPW)], idx_v)
        for t in range(NSUB):
            pltpu.async_copy(
                flat_hbm.at[idx_v.at[pl.ds(t * SUB, SUB)]],
                val_v.at[pl.ds(t * SUB, SUB)],
                sem,
            ).wait()
        pltpu.sync_copy(val_v, bC_hbm.at[pl.ds(c * NPAD + base, BPW)])


_sc_stage = functools.partial(
    pl.kernel,
    mesh=plsc.VectorSubcoreMesh(core_axis_name="c", subcore_axis_name="s"),
    out_type=[
        jax.ShapeDtypeStruct((4 * NPAD,), jnp.float32),
        jax.ShapeDtypeStruct((NPAD,), jnp.float32),
    ],
    scratch_types=[
        pltpu.VMEM((BPW,), jnp.int32),
        pltpu.VMEM((BPW,), jnp.float32),
        pltpu.VMEM((BPW,), jnp.int32),
        pltpu.SemaphoreType.DMA,
    ],
)(_sc_stage_kernel)


CH = 512
CPB = CH // BLK


def _nms_block_kernel(bT_ref, bC_ref, sC_ref, out_ref, keep_ref):
    k = pl.program_id(0)

    @pl.when(k == 0)
    def _init():
        keep_ref[...] = jnp.zeros_like(keep_ref)

    cols = pl.ds(k * BLK, BLK)
    y1b, x1b = bC_ref[0:1, cols], bC_ref[1:2, cols]
    y2b, x2b = bC_ref[2:3, cols], bC_ref[3:4, cols]
    area_b = (y2b - y1b) * (x2b - x1b)

    def chunk_body(m, hits):
        rows = pl.ds(m * CH, CH)
        cb = bT_ref[rows, :]
        y1a, x1a = cb[:, 0:1], cb[:, 1:2]
        y2a, x2a = cb[:, 2:3], cb[:, 3:4]
        area_a = (y2a - y1a) * (x2a - x1a)
        yy1 = jnp.maximum(y1a, y1b)
        xx1 = jnp.maximum(x1a, x1b)
        yy2 = jnp.minimum(y2a, y2b)
        xx2 = jnp.minimum(x2a, x2b)
        ih = jnp.maximum(yy2 - yy1, 0.0)
        iw = jnp.maximum(xx2 - xx1, 0.0)
        inter = ih * iw
        union = area_a + area_b - inter
        iou = inter / (union + 1e-8)
        supT = (iou > IOU_THRESH).astype(jnp.float32)
        return hits + lax.dot_general(
            keep_ref[0:1, rows], supT, (((1,), (0,)), ((), ())),
            preferred_element_type=jnp.float32,
        )

    hits = lax.fori_loop(
        0, k // CPB + 1, chunk_body, jnp.zeros((1, BLK), jnp.float32)
    )
    kb0 = (hits == 0.0).astype(jnp.float32)

    rb = bT_ref[pl.ds(k * BLK, BLK), :]
    y1r, x1r = rb[:, 0:1], rb[:, 1:2]
    y2r, x2r = rb[:, 2:3], rb[:, 3:4]
    area_r = (y2r - y1r) * (x2r - x1r)
    byy1 = jnp.maximum(y1r, y1b)
    bxx1 = jnp.maximum(x1r, x1b)
    byy2 = jnp.minimum(y2r, y2b)
    bxx2 = jnp.minimum(x2r, x2b)
    bih = jnp.maximum(byy2 - byy1, 0.0)
    biw = jnp.maximum(bxx2 - bxx1, 0.0)
    binter = bih * biw
    bunion = area_r + area_b - binter
    biou = binter / (bunion + 1e-8)
    rowi = lax.broadcasted_iota(jnp.int32, (BLK, BLK), 0)
    coli = lax.broadcasted_iota(jnp.int32, (BLK, BLK), 1)
    locU = ((biou > IOU_THRESH) & (coli > rowi)).astype(jnp.float32)

    def cond(c):
        return jnp.logical_not(c[1])

    def body(c):
        kb, _ = c
        h = lax.dot_general(
            kb, locU, (((1,), (0,)), ((), ())),
            preferred_element_type=jnp.float32,
        )
        kb2 = kb0 * (h == 0.0).astype(jnp.float32)
        return kb2, jnp.all(kb2 == kb)

    kb, _ = lax.while_loop(cond, body, (kb0, False))

    keep_ref[0:1, cols] = kb

    bcols = bC_ref[:, cols]
    srow = sC_ref[...]
    out_ref[...] = jnp.concatenate(
        [bcols * kb, srow * kb, jnp.zeros((3, BLK), jnp.float32)], axis=0
    )


@jax.jit
def kernel(boxes, scores):
    order = jnp.argsort(-scores)
    ordp = jnp.concatenate(
        [order, jnp.full((NPAD - N,), N, jnp.int32)]
    ).astype(jnp.int32)
    flat = jnp.concatenate([boxes.reshape(-1), jnp.zeros((4,), jnp.float32)])
    sc_tab = jnp.concatenate([scores, jnp.zeros((1,), jnp.float32)])
    idx = (ordp[None, :] * 4 + jnp.arange(4, dtype=jnp.int32)[:, None]).reshape(-1)

    bCf, sCf = _sc_stage(flat, sc_tab, idx, ordp)
    bC = bCf.reshape(4, NPAD)
    bT = bC.T
    sC = sCf.reshape(1, NPAD)

    outT = pl.pallas_call(
        _nms_block_kernel,
        grid=(NBLK,),
        in_specs=[
            pl.BlockSpec((NPAD, 4), lambda k: (0, 0)),
            pl.BlockSpec((4, NPAD), lambda k: (0, 0)),
            pl.BlockSpec((1, BLK), lambda k: (0, k)),
        ],
        out_specs=pl.BlockSpec((8, BLK), lambda k: (0, k)),
        out_shape=jax.ShapeDtypeStruct((8, NPAD), jnp.float32),
        scratch_shapes=[
            pltpu.VMEM((1, NPAD), jnp.float32),
        ],
    )(bT, bC, sC)

    return outT[:5, :N].T


# speedup vs baseline: 96.9783x; 1.8450x over previous
import functools

import jax
import jax.numpy as jnp
from jax import lax
from jax.experimental import pallas as pl
from jax.experimental.pallas import tpu as pltpu
from jax.experimental.pallas import tpu_sc as plsc

N = 5000
BLK = 128
NPAD = 5120
NBLK = NPAD // BLK
IOU_THRESH = 0.5

NW = 32
BPW = NPAD // NW
SUB = 80
NSUB = BPW // SUB


def _sc_stage_kernel(
    flat_hbm, scores_hbm, idx_hbm, ordp_hbm,
    bC_hbm, sC_hbm,
    idx_v, val_v, ord_v, sem,
):
    w = lax.axis_index("s") * 2 + lax.axis_index("c")
    base = w * BPW

    pltpu.sync_copy(ordp_hbm.at[pl.ds(base, BPW)], ord_v)
    for t in range(NSUB):
        pltpu.async_copy(
            scores_hbm.at[ord_v.at[pl.ds(t * SUB, SUB)]],
            val_v.at[pl.ds(t * SUB, SUB)],
            sem,
        ).wait()
    pltpu.sync_copy(val_v, sC_hbm.at[pl.ds(base, BPW)])

    for c in range(4):
        pltpu.sync_copy(idx_hbm.at[pl.ds(c * NPAD + base, BPW)], idx_v)
        for t in range(NSUB):
            pltpu.async_copy(
                flat_hbm.at[idx_v.at[pl.ds(t * SUB, SUB)]],
                val_v.at[pl.ds(t * SUB, SUB)],
                sem,
            ).wait()
        pltpu.sync_copy(val_v, bC_hbm.at[pl.ds(c * NPAD + base, BPW)])


_sc_stage = functools.partial(
    pl.kernel,
    mesh=plsc.VectorSubcoreMesh(core_axis_name="c", subcore_axis_name="s"),
    out_type=[
        jax.ShapeDtypeStruct((4 * NPAD,), jnp.float32),
        jax.ShapeDtypeStruct((NPAD,), jnp.float32),
    ],
    scratch_types=[
        pltpu.VMEM((BPW,), jnp.int32),
        pltpu.VMEM((BPW,), jnp.float32),
        pltpu.VMEM((BPW,), jnp.int32),
        pltpu.SemaphoreType.DMA,
    ],
)(_sc_stage_kernel)


CH = 512
CPB = CH // BLK


def _nms_block_kernel(
    bT_ref, bC_ref, sC_ref, out_ref,
    keep_ref, y1P, x1P, y2P, x2P, aP,
):
    k = pl.program_id(0)

    @pl.when(k == 0)
    def _init():
        keep_ref[...] = jnp.zeros_like(keep_ref)
        y1c = jnp.broadcast_to(bT_ref[:, 0:1], (NPAD, BLK))
        x1c = jnp.broadcast_to(bT_ref[:, 1:2], (NPAD, BLK))
        y2c = jnp.broadcast_to(bT_ref[:, 2:3], (NPAD, BLK))
        x2c = jnp.broadcast_to(bT_ref[:, 3:4], (NPAD, BLK))
        y1P[...] = y1c
        x1P[...] = x1c
        y2P[...] = y2c
        x2P[...] = x2c
        aP[...] = (y2c - y1c) * (x2c - x1c)

    cols = pl.ds(k * BLK, BLK)
    y1b, x1b = bC_ref[0:1, cols], bC_ref[1:2, cols]
    y2b, x2b = bC_ref[2:3, cols], bC_ref[3:4, cols]
    area_b = (y2b - y1b) * (x2b - x1b)

    def chunk_body(m, hits):
        rows = pl.ds(m * CH, CH)
        yy1 = jnp.maximum(y1P[rows, :], y1b)
        xx1 = jnp.maximum(x1P[rows, :], x1b)
        yy2 = jnp.minimum(y2P[rows, :], y2b)
        xx2 = jnp.minimum(x2P[rows, :], x2b)
        ih = jnp.maximum(yy2 - yy1, 0.0)
        iw = jnp.maximum(xx2 - xx1, 0.0)
        inter = ih * iw
        union = aP[rows, :] + area_b - inter
        iou = inter / (union + 1e-8)
        supT = (iou > IOU_THRESH).astype(jnp.float32)
        return hits + lax.dot_general(
            keep_ref[0:1, rows], supT, (((1,), (0,)), ((), ())),
            preferred_element_type=jnp.float32,
        )

    hits = lax.fori_loop(
        0, k // CPB + 1, chunk_body, jnp.zeros((1, BLK), jnp.float32)
    )
    kb0 = (hits == 0.0).astype(jnp.float32)

    brows = pl.ds(k * BLK, BLK)
    area_r = aP[brows, :]
    byy1 = jnp.maximum(y1P[brows, :], y1b)
    bxx1 = jnp.maximum(x1P[brows, :], x1b)
    byy2 = jnp.minimum(y2P[brows, :], y2b)
    bxx2 = jnp.minimum(x2P[brows, :], x2b)
    bih = jnp.maximum(byy2 - byy1, 0.0)
    biw = jnp.maximum(bxx2 - bxx1, 0.0)
    binter = bih * biw
    bunion = area_r + area_b - binter
    biou = binter / (bunion + 1e-8)
    rowi = lax.broadcasted_iota(jnp.int32, (BLK, BLK), 0)
    coli = lax.broadcasted_iota(jnp.int32, (BLK, BLK), 1)
    locU = ((biou > IOU_THRESH) & (coli > rowi)).astype(jnp.float32)

    def cond(c):
        return jnp.logical_not(c[1])

    def body(c):
        kb, _ = c
        h = lax.dot_general(
            kb, locU, (((1,), (0,)), ((), ())),
            preferred_element_type=jnp.float32,
        )
        kb2 = kb0 * (h == 0.0).astype(jnp.float32)
        return kb2, jnp.all(kb2 == kb)

    kb, _ = lax.while_loop(cond, body, (kb0, False))

    keep_ref[0:1, cols] = kb

    bcols = bC_ref[:, cols]
    srow = sC_ref[...]
    out_ref[...] = jnp.concatenate(
        [bcols * kb, srow * kb, jnp.zeros((3, BLK), jnp.float32)], axis=0
    )


@jax.jit
def kernel(boxes, scores):
    order = jnp.argsort(-scores)
    ordp = jnp.concatenate(
        [order, jnp.full((NPAD - N,), N, jnp.int32)]
    ).astype(jnp.int32)
    flat = jnp.concatenate([boxes.reshape(-1), jnp.zeros((4,), jnp.float32)])
    sc_tab = jnp.concatenate([scores, jnp.zeros((1,), jnp.float32)])
    idx = (ordp[None, :] * 4 + jnp.arange(4, dtype=jnp.int32)[:, None]).reshape(-1)

    bCf, sCf = _sc_stage(flat, sc_tab, idx, ordp)
    bC = bCf.reshape(4, NPAD)
    bT = bC.T
    sC = sCf.reshape(1, NPAD)

    outT = pl.pallas_call(
        _nms_block_kernel,
        grid=(NBLK,),
        in_specs=[
            pl.BlockSpec((NPAD, 4), lambda k: (0, 0)),
            pl.BlockSpec((4, NPAD), lambda k: (0, 0)),
            pl.BlockSpec((1, BLK), lambda k: (0, k)),
        ],
        out_specs=pl.BlockSpec((8, BLK), lambda k: (0, k)),
        out_shape=jax.ShapeDtypeStruct((8, NPAD), jnp.float32),
        scratch_shapes=[
            pltpu.VMEM((1, NPAD), jnp.float32),
            pltpu.VMEM((NPAD, BLK), jnp.float32),
            pltpu.VMEM((NPAD, BLK), jnp.float32),
            pltpu.VMEM((NPAD, BLK), jnp.float32),
            pltpu.VMEM((NPAD, BLK), jnp.float32),
            pltpu.VMEM((NPAD, BLK), jnp.float32),
        ],
    )(bT, bC, sC)

    return outT[:5, :N].T


# speedup vs baseline: 105.7599x; 1.0906x over previous
import functools

import jax
import jax.numpy as jnp
from jax import lax
from jax.experimental import pallas as pl
from jax.experimental.pallas import tpu as pltpu
from jax.experimental.pallas import tpu_sc as plsc

N = 5000
BLK = 128
NPAD = 5120
NBLK = NPAD // BLK
IOU_THRESH = 0.5

NW = 32
BPW = NPAD // NW
SUB = 80
NSUB = BPW // SUB


def _sc_stage_kernel(
    flat_hbm, scores_hbm, idx_hbm, ordp_hbm,
    bC_hbm, sC_hbm,
    idx_v, val_v, ord_v, sem,
):
    w = lax.axis_index("s") * 2 + lax.axis_index("c")
    base = w * BPW

    pltpu.sync_copy(ordp_hbm.at[pl.ds(base, BPW)], ord_v)
    for t in range(NSUB):
        pltpu.async_copy(
            scores_hbm.at[ord_v.at[pl.ds(t * SUB, SUB)]],
            val_v.at[pl.ds(t * SUB, SUB)],
            sem,
        ).wait()
    pltpu.sync_copy(val_v, sC_hbm.at[pl.ds(base, BPW)])

    for c in range(4):
        pltpu.sync_copy(idx_hbm.at[pl.ds(c * NPAD + base, BPW)], idx_v)
        for t in range(NSUB):
            pltpu.async_copy(
                flat_hbm.at[idx_v.at[pl.ds(t * SUB, SUB)]],
                val_v.at[pl.ds(t * SUB, SUB)],
                sem,
            ).wait()
        pltpu.sync_copy(val_v, bC_hbm.at[pl.ds(c * NPAD + base, BPW)])


_sc_stage = functools.partial(
    pl.kernel,
    mesh=plsc.VectorSubcoreMesh(core_axis_name="c", subcore_axis_name="s"),
    out_type=[
        jax.ShapeDtypeStruct((4 * NPAD,), jnp.float32),
        jax.ShapeDtypeStruct((NPAD,), jnp.float32),
    ],
    scratch_types=[
        pltpu.VMEM((BPW,), jnp.int32),
        pltpu.VMEM((BPW,), jnp.float32),
        pltpu.VMEM((BPW,), jnp.int32),
        pltpu.SemaphoreType.DMA,
    ],
)(_sc_stage_kernel)


CH = 1024
CPB = CH // BLK


def _nms_block_kernel(
    bT_ref, bC_ref, sC_ref, out_ref,
    keep_ref, y1P, x1P, y2P, x2P, aP,
):
    k = pl.program_id(0)

    @pl.when(k == 0)
    def _init():
        keep_ref[...] = jnp.zeros_like(keep_ref)
        y1c = jnp.broadcast_to(bT_ref[:, 0:1], (NPAD, BLK))
        x1c = jnp.broadcast_to(bT_ref[:, 1:2], (NPAD, BLK))
        y2c = jnp.broadcast_to(bT_ref[:, 2:3], (NPAD, BLK))
        x2c = jnp.broadcast_to(bT_ref[:, 3:4], (NPAD, BLK))
        y1P[...] = y1c
        x1P[...] = x1c
        y2P[...] = y2c
        x2P[...] = x2c
        aP[...] = (y2c - y1c) * (x2c - x1c)

    cols = pl.ds(k * BLK, BLK)
    y1b, x1b = bC_ref[0:1, cols], bC_ref[1:2, cols]
    y2b, x2b = bC_ref[2:3, cols], bC_ref[3:4, cols]
    area_b = (y2b - y1b) * (x2b - x1b)

    def chunk_body(m, hits):
        rows = pl.ds(m * CH, CH)
        yy1 = jnp.maximum(y1P[rows, :], y1b)
        xx1 = jnp.maximum(x1P[rows, :], x1b)
        yy2 = jnp.minimum(y2P[rows, :], y2b)
        xx2 = jnp.minimum(x2P[rows, :], x2b)
        ih = jnp.maximum(yy2 - yy1, 0.0)
        iw = jnp.maximum(xx2 - xx1, 0.0)
        inter = ih * iw
        union = aP[rows, :] + area_b - inter
        iou = inter / (union + 1e-8)
        supT = (iou > IOU_THRESH).astype(jnp.float32)
        return hits + lax.dot_general(
            keep_ref[0:1, rows], supT, (((1,), (0,)), ((), ())),
            preferred_element_type=jnp.float32,
        )

    hits = lax.fori_loop(
        0, k // CPB + 1, chunk_body, jnp.zeros((1, BLK), jnp.float32)
    )
    kb0 = (hits == 0.0).astype(jnp.float32)

    brows = pl.ds(k * BLK, BLK)
    area_r = aP[brows, :]
    byy1 = jnp.maximum(y1P[brows, :], y1b)
    bxx1 = jnp.maximum(x1P[brows, :], x1b)
    byy2 = jnp.minimum(y2P[brows, :], y2b)
    bxx2 = jnp.minimum(x2P[brows, :], x2b)
    bih = jnp.maximum(byy2 - byy1, 0.0)
    biw = jnp.maximum(bxx2 - bxx1, 0.0)
    binter = bih * biw
    bunion = area_r + area_b - binter
    biou = binter / (bunion + 1e-8)
    rowi = lax.broadcasted_iota(jnp.int32, (BLK, BLK), 0)
    coli = lax.broadcasted_iota(jnp.int32, (BLK, BLK), 1)
    locU = ((biou > IOU_THRESH) & (coli > rowi)).astype(jnp.float32)

    def cond(c):
        return jnp.logical_not(c[1])

    def body(c):
        kb, _ = c
        h = lax.dot_general(
            kb, locU, (((1,), (0,)), ((), ())),
            preferred_element_type=jnp.float32,
        )
        kb2 = kb0 * (h == 0.0).astype(jnp.float32)
        return kb2, jnp.all(kb2 == kb)

    kb, _ = lax.while_loop(cond, body, (kb0, False))

    keep_ref[0:1, cols] = kb

    bcols = bC_ref[:, cols]
    srow = sC_ref[...]
    out_ref[...] = jnp.concatenate(
        [bcols * kb, srow * kb, jnp.zeros((3, BLK), jnp.float32)], axis=0
    )


@jax.jit
def kernel(boxes, scores):
    order = jnp.argsort(-scores)
    ordp = jnp.concatenate(
        [order, jnp.full((NPAD - N,), N, jnp.int32)]
    ).astype(jnp.int32)
    flat = jnp.concatenate([boxes.reshape(-1), jnp.zeros((4,), jnp.float32)])
    sc_tab = jnp.concatenate([scores, jnp.zeros((1,), jnp.float32)])
    idx = (ordp[None, :] * 4 + jnp.arange(4, dtype=jnp.int32)[:, None]).reshape(-1)

    bCf, sCf = _sc_stage(flat, sc_tab, idx, ordp)
    bC = bCf.reshape(4, NPAD)
    bT = bC.T
    sC = sCf.reshape(1, NPAD)

    outT = pl.pallas_call(
        _nms_block_kernel,
        grid=(NBLK,),
        in_specs=[
            pl.BlockSpec((NPAD, 4), lambda k: (0, 0)),
            pl.BlockSpec((4, NPAD), lambda k: (0, 0)),
            pl.BlockSpec((1, BLK), lambda k: (0, k)),
        ],
        out_specs=pl.BlockSpec((8, BLK), lambda k: (0, k)),
        out_shape=jax.ShapeDtypeStruct((8, NPAD), jnp.float32),
        scratch_shapes=[
            pltpu.VMEM((1, NPAD), jnp.float32),
            pltpu.VMEM((NPAD, BLK), jnp.float32),
            pltpu.VMEM((NPAD, BLK), jnp.float32),
            pltpu.VMEM((NPAD, BLK), jnp.float32),
            pltpu.VMEM((NPAD, BLK), jnp.float32),
            pltpu.VMEM((NPAD, BLK), jnp.float32),
        ],
    )(bT, bC, sC)

    return outT[:5, :N].T
